# baseline (speedup 1.0000x reference)
import jax
import jax.numpy as jnp
from jax.experimental import pallas as pl
from jax.experimental.pallas import tpu as pltpu

_N_GROUP = 8
_TOPK_GROUP = 4
_TOP_K = 8
_SCALE = 2.5
_NEG = -1e30


def _gate_kernel(x_ref, wt_ref, b_ref, idx_ref, w_ref):
    x = x_ref[...]
    wt = wt_ref[...]
    logits = jnp.dot(x, wt, preferred_element_type=jnp.float32)
    scores = jax.nn.sigmoid(logits)
    sfc = scores + b_ref[...]

    bt, e = scores.shape
    spg = e // _N_GROUP
    lane = jax.lax.broadcasted_iota(jnp.int32, (bt, e), 1)
    lig = lane % spg
    gid = lane // spg

    def rot_in_group(v, k):
        a = pltpu.roll(v, e - k, axis=1)
        b = pltpu.roll(v, spg - k, axis=1)
        return jnp.where(lig < spg - k, a, b)

    def group_reduce_bcast(v, op):
        k = 1
        while k < spg:
            v = op(v, rot_in_group(v, k))
            k *= 2
        return v

    m1 = group_reduce_bcast(sfc, jnp.maximum)
    i1 = group_reduce_bcast(jnp.where(sfc == m1, lane, e), jnp.minimum)
    m2 = group_reduce_bcast(jnp.where(lane == i1, _NEG, sfc), jnp.maximum)
    gs_full = m1 + m2

    rank = jnp.zeros((bt, e), dtype=jnp.int32)
    for k in range(1, _N_GROUP):
        gv = pltpu.roll(gs_full, e - spg * k, axis=1)
        ogid = (gid + k) % _N_GROUP
        beats = (gv > gs_full) | ((gv == gs_full) & (ogid < gid))
        rank = rank + beats.astype(jnp.int32)
    gmask = rank < _TOPK_GROUP

    tmp = jnp.where(gmask, sfc, 0.0)

    col = jax.lax.broadcasted_iota(jnp.int32, (bt, _TOP_K), 1)
    acc_idx = jnp.zeros((bt, _TOP_K), dtype=jnp.int32)
    acc_w = jnp.zeros((bt, _TOP_K), dtype=jnp.float32)
    t = tmp
    for k in range(_TOP_K):
        m = jnp.max(t, axis=1, keepdims=True)
        i = jnp.min(jnp.where(t == m, lane, e), axis=1, keepdims=True)
        onehot = lane == i
        wk = jnp.max(jnp.where(onehot, scores, _NEG), axis=1, keepdims=True)
        acc_idx = jnp.where(col == k, i, acc_idx)
        acc_w = jnp.where(col == k, wk, acc_w)
        t = jnp.where(onehot, _NEG, t)

    denom = jnp.sum(acc_w, axis=1, keepdims=True) + 1e-20
    idx_ref[...] = acc_idx
    w_ref[...] = acc_w / denom * _SCALE


def kernel(hidden_states, weight, e_score_correction_bias):
    bsz, seq, h = hidden_states.shape
    n_experts = weight.shape[0]
    t = bsz * seq
    bt = 256

    x2 = hidden_states.reshape(t, h)
    wt = weight.astype(jnp.float32).T
    b2 = e_score_correction_bias.reshape(1, n_experts).astype(jnp.float32)

    idx, w = pl.pallas_call(
        _gate_kernel,
        grid=(t // bt,),
        in_specs=[
            pl.BlockSpec((bt, h), lambda i: (i, 0)),
            pl.BlockSpec((h, n_experts), lambda i: (0, 0)),
            pl.BlockSpec((1, n_experts), lambda i: (0, 0)),
        ],
        out_specs=[
            pl.BlockSpec((bt, _TOP_K), lambda i: (i, 0)),
            pl.BlockSpec((bt, _TOP_K), lambda i: (i, 0)),
        ],
        out_shape=[
            jax.ShapeDtypeStruct((t, _TOP_K), jnp.int32),
            jax.ShapeDtypeStruct((t, _TOP_K), jnp.float32),
        ],
        compiler_params=pltpu.CompilerParams(
            dimension_semantics=("arbitrary",),
        ),
    )(x2, wt, b2)
    return idx, w


# speedup vs baseline: 2.7932x; 2.7932x over previous
import jax
import jax.numpy as jnp
from jax.experimental import pallas as pl
from jax.experimental.pallas import tpu as pltpu

_N_GROUP = 8
_TOPK_GROUP = 4
_TOP_K = 8
_SCALE = 2.5
_NEG = -1e30


def _gate_kernel(x_ref, w_ref, b_ref, idx_ref, w_out_ref):
    x = x_ref[...]
    w = w_ref[...]
    logits_t = jax.lax.dot_general(
        w, x, (((1,), (1,)), ((), ())),
        preferred_element_type=jnp.float32)
    scores_t = jax.nn.sigmoid(logits_t)
    sfc = scores_t + b_ref[...]

    e, bt = sfc.shape
    spg = e // _N_GROUP

    g3 = sfc.reshape(_N_GROUP, spg, bt)
    m1 = jnp.max(g3, axis=1, keepdims=True)
    m1b = jnp.broadcast_to(m1, g3.shape)
    eq = g3 == m1b
    cnt = jnp.sum(eq.astype(jnp.float32), axis=1, keepdims=True)
    strict = jnp.max(jnp.where(eq, _NEG, g3), axis=1, keepdims=True)
    m2 = jnp.where(cnt >= 2.0, m1, strict)
    gs = m1 + m2

    growf = jax.lax.broadcasted_iota(jnp.int32, (_N_GROUP, 1, bt), 0).astype(jnp.float32)
    gidf = (jax.lax.broadcasted_iota(jnp.int32, (e, bt), 0) // spg).astype(jnp.float32)
    t8 = gs
    gmask = jnp.zeros((e, bt), dtype=jnp.bool_)
    for _ in range(_TOPK_GROUP):
        m = jnp.max(t8, axis=0, keepdims=True)
        fi = jnp.min(jnp.where(t8 == m, growf, float(_N_GROUP)),
                     axis=0, keepdims=True)
        fi2 = fi.reshape(1, bt)
        gmask = gmask | (gidf == fi2)
        t8 = jnp.where(growf == fi, _NEG, t8)

    tmp = jnp.where(gmask, sfc, 0.0)

    frow = jax.lax.broadcasted_iota(jnp.int32, (e, bt), 0).astype(jnp.float32)
    row8 = jax.lax.broadcasted_iota(jnp.int32, (_TOP_K, bt), 0).astype(jnp.float32)
    acc_i = jnp.zeros((_TOP_K, bt), dtype=jnp.float32)
    acc_w = jnp.zeros((_TOP_K, bt), dtype=jnp.float32)
    t = tmp
    for k in range(_TOP_K):
        m = jnp.max(t, axis=0, keepdims=True)
        fi = jnp.min(jnp.where(t == m, frow, float(e)),
                     axis=0, keepdims=True)
        acc_i = jnp.where(row8 == float(k), fi, acc_i)
        acc_w = jnp.where(row8 == float(k), m, acc_w)
        t = jnp.where(frow == fi, _NEG, t)

    denom = jnp.sum(acc_w, axis=0, keepdims=True) + 1e-20
    w_out = acc_w * (_SCALE / denom)

    idx_ref[...] = acc_i.astype(jnp.int32).T
    w_out_ref[...] = w_out.T


def kernel(hidden_states, weight, e_score_correction_bias):
    bsz, seq, h = hidden_states.shape
    n_experts = weight.shape[0]
    t = bsz * seq
    bt = 256

    x2 = hidden_states.reshape(t, h)
    w = weight.astype(jnp.float32)
    b2 = e_score_correction_bias.reshape(n_experts, 1).astype(jnp.float32)

    idx, wts = pl.pallas_call(
        _gate_kernel,
        grid=(t // bt,),
        in_specs=[
            pl.BlockSpec((bt, h), lambda i: (i, 0)),
            pl.BlockSpec((n_experts, h), lambda i: (0, 0)),
            pl.BlockSpec((n_experts, 1), lambda i: (0, 0)),
        ],
        out_specs=[
            pl.BlockSpec((bt, _TOP_K), lambda i: (i, 0)),
            pl.BlockSpec((bt, _TOP_K), lambda i: (i, 0)),
        ],
        out_shape=[
            jax.ShapeDtypeStruct((t, _TOP_K), jnp.int32),
            jax.ShapeDtypeStruct((t, _TOP_K), jnp.float32),
        ],
        compiler_params=pltpu.CompilerParams(
            dimension_semantics=("arbitrary",),
        ),
    )(x2, w, b2)
    return idx, wts


# speedup vs baseline: 3.2846x; 1.1759x over previous
import jax
import jax.numpy as jnp
from jax.experimental import pallas as pl
from jax.experimental.pallas import tpu as pltpu

_N_GROUP = 8
_TOPK_GROUP = 4
_TOP_K = 8
_SCALE = 2.5
_NEG = -1e30


def _gate_kernel(x_ref, w_ref, b_ref, idx_ref, w_out_ref):
    x = x_ref[...]
    w = w_ref[...]
    logits_t = jax.lax.dot_general(
        w, x, (((1,), (1,)), ((), ())),
        preferred_element_type=jnp.float32)
    scores_t = jax.nn.sigmoid(logits_t)
    sfc = scores_t + b_ref[...]

    e, bt = sfc.shape
    spg = e // _N_GROUP

    g3 = sfc.reshape(_N_GROUP, spg, bt)
    m1 = jnp.max(g3, axis=1, keepdims=True)
    m1b = jnp.broadcast_to(m1, g3.shape)
    eq = g3 == m1b
    cnt = jnp.sum(eq.astype(jnp.float32), axis=1, keepdims=True)
    strict = jnp.max(jnp.where(eq, _NEG, g3), axis=1, keepdims=True)
    m2 = jnp.where(cnt >= 2.0, m1, strict)
    gs = m1 + m2

    growf = jax.lax.broadcasted_iota(jnp.int32, (_N_GROUP, 1, bt), 0).astype(jnp.float32)
    gidf = (jax.lax.broadcasted_iota(jnp.int32, (e, bt), 0) // spg).astype(jnp.float32)
    t8 = gs
    gmask = jnp.zeros((e, bt), dtype=jnp.bool_)
    for _ in range(_TOPK_GROUP):
        m = jnp.max(t8, axis=0, keepdims=True)
        fi = jnp.min(jnp.where(t8 == m, growf, float(_N_GROUP)),
                     axis=0, keepdims=True)
        fi2 = fi.reshape(1, bt)
        gmask = gmask | (gidf == fi2)
        t8 = jnp.where(growf == fi, _NEG, t8)

    tmp = jnp.where(gmask, sfc, 0.0)

    frow = jax.lax.broadcasted_iota(jnp.int32, (e, bt), 0).astype(jnp.float32)
    row8 = jax.lax.broadcasted_iota(jnp.int32, (_TOP_K, bt), 0).astype(jnp.float32)
    acc_i = jnp.zeros((_TOP_K, bt), dtype=jnp.float32)
    acc_w = jnp.zeros((_TOP_K, bt), dtype=jnp.float32)
    t = tmp
    for k in range(_TOP_K):
        m = jnp.max(t, axis=0, keepdims=True)
        fi = jnp.min(jnp.where(t == m, frow, float(e)),
                     axis=0, keepdims=True)
        acc_i = jnp.where(row8 == float(k), fi, acc_i)
        acc_w = jnp.where(row8 == float(k), m, acc_w)
        t = jnp.where(frow == fi, _NEG, t)

    denom = jnp.sum(acc_w, axis=0, keepdims=True) + 1e-20
    w_out = acc_w * (_SCALE / denom)

    idx_ref[...] = acc_i.astype(jnp.int32).T
    w_out_ref[...] = w_out.T


def kernel(hidden_states, weight, e_score_correction_bias):
    bsz, seq, h = hidden_states.shape
    n_experts = weight.shape[0]
    t = bsz * seq
    bt = 512

    x2 = hidden_states.reshape(t, h)
    w = weight.astype(jnp.float32)
    b2 = e_score_correction_bias.reshape(n_experts, 1).astype(jnp.float32)

    idx, wts = pl.pallas_call(
        _gate_kernel,
        grid=(t // bt,),
        in_specs=[
            pl.BlockSpec((bt, h), lambda i: (i, 0)),
            pl.BlockSpec((n_experts, h), lambda i: (0, 0)),
            pl.BlockSpec((n_experts, 1), lambda i: (0, 0)),
        ],
        out_specs=[
            pl.BlockSpec((bt, _TOP_K), lambda i: (i, 0)),
            pl.BlockSpec((bt, _TOP_K), lambda i: (i, 0)),
        ],
        out_shape=[
            jax.ShapeDtypeStruct((t, _TOP_K), jnp.int32),
            jax.ShapeDtypeStruct((t, _TOP_K), jnp.float32),
        ],
        compiler_params=pltpu.CompilerParams(
            dimension_semantics=("arbitrary",),
        ),
    )(x2, w, b2)
    return idx, wts


# speedup vs baseline: 3.5089x; 1.0683x over previous
import jax
import jax.numpy as jnp
from jax.experimental import pallas as pl
from jax.experimental.pallas import tpu as pltpu

_N_GROUP = 8
_TOPK_GROUP = 4
_TOP_K = 8
_SCALE = 2.5
_NEG = -1e30


def _gate_kernel(x_ref, w_ref, b_ref, idx_ref, w_out_ref):
    x = x_ref[...]
    w = w_ref[...]
    logits_t = jax.lax.dot_general(
        w, x, (((1,), (1,)), ((), ())),
        preferred_element_type=jnp.float32)
    scores_t = jax.nn.sigmoid(logits_t)
    sfc = scores_t + b_ref[...]

    e, bt = sfc.shape
    spg = e // _N_GROUP

    g3 = sfc.reshape(_N_GROUP, spg, bt)
    m1 = jnp.max(g3, axis=1, keepdims=True)
    m1b = jnp.broadcast_to(m1, g3.shape)
    eq = g3 == m1b
    cnt = jnp.sum(eq.astype(jnp.float32), axis=1, keepdims=True)
    strict = jnp.max(jnp.where(eq, _NEG, g3), axis=1, keepdims=True)
    m2 = jnp.where(cnt >= 2.0, m1, strict)
    gs = m1 + m2

    growf = jax.lax.broadcasted_iota(jnp.int32, (_N_GROUP, 1, bt), 0).astype(jnp.float32)
    gidf = (jax.lax.broadcasted_iota(jnp.int32, (e, bt), 0) // spg).astype(jnp.float32)
    t8 = gs
    gmask = jnp.zeros((e, bt), dtype=jnp.bool_)
    for _ in range(_TOPK_GROUP):
        m = jnp.max(t8, axis=0, keepdims=True)
        fi = jnp.min(jnp.where(t8 == m, growf, float(_N_GROUP)),
                     axis=0, keepdims=True)
        fi2 = fi.reshape(1, bt)
        gmask = gmask | (gidf == fi2)
        t8 = jnp.where(growf == fi, _NEG, t8)

    tmp = jnp.where(gmask, sfc, 0.0)

    frow = jax.lax.broadcasted_iota(jnp.int32, (e, bt), 0).astype(jnp.float32)
    row8 = jax.lax.broadcasted_iota(jnp.int32, (_TOP_K, bt), 0).astype(jnp.float32)
    acc_i = jnp.zeros((_TOP_K, bt), dtype=jnp.float32)
    acc_w = jnp.zeros((_TOP_K, bt), dtype=jnp.float32)
    t = tmp
    for k in range(_TOP_K):
        m = jnp.max(t, axis=0, keepdims=True)
        fi = jnp.min(jnp.where(t == m, frow, float(e)),
                     axis=0, keepdims=True)
        acc_i = jnp.where(row8 == float(k), fi, acc_i)
        acc_w = jnp.where(row8 == float(k), m, acc_w)
        t = jnp.where(frow == fi, _NEG, t)

    denom = jnp.sum(acc_w, axis=0, keepdims=True) + 1e-20
    w_out = acc_w * (_SCALE / denom)

    idx_ref[...] = acc_i.astype(jnp.int32).T
    w_out_ref[...] = w_out.T


def kernel(hidden_states, weight, e_score_correction_bias):
    bsz, seq, h = hidden_states.shape
    n_experts = weight.shape[0]
    t = bsz * seq
    bt = 1024

    x2 = hidden_states.reshape(t, h)
    w = weight.astype(jnp.float32)
    b2 = e_score_correction_bias.reshape(n_experts, 1).astype(jnp.float32)

    idx, wts = pl.pallas_call(
        _gate_kernel,
        grid=(t // bt,),
        in_specs=[
            pl.BlockSpec((bt, h), lambda i: (i, 0)),
            pl.BlockSpec((n_experts, h), lambda i: (0, 0)),
            pl.BlockSpec((n_experts, 1), lambda i: (0, 0)),
        ],
        out_specs=[
            pl.BlockSpec((bt, _TOP_K), lambda i: (i, 0)),
            pl.BlockSpec((bt, _TOP_K), lambda i: (i, 0)),
        ],
        out_shape=[
            jax.ShapeDtypeStruct((t, _TOP_K), jnp.int32),
            jax.ShapeDtypeStruct((t, _TOP_K), jnp.float32),
        ],
        compiler_params=pltpu.CompilerParams(
            dimension_semantics=("arbitrary",),
        ),
    )(x2, w, b2)
    return idx, wts
